# Initial kernel scaffold; baseline (speedup 1.0000x reference)
import jax
import jax.numpy as jnp
from jax.experimental import pallas as pl


def kernel(continuous_attrs, categorical_attrs, wide_W, wide_b, adep_tab, ades_tab, cluster_tab, fc1_W, fc1_b, fc2_W, fc2_b):
    raise NotImplementedError("write your pallas kernel here")


# speedup vs baseline: 3.9295x; 3.9295x over previous
import functools

import jax
import jax.numpy as jnp
from jax import lax
from jax.experimental import pallas as pl
from jax.experimental.pallas import tpu as pltpu
from jax.experimental.pallas import tpu_sc as plsc

_B = 16384
_CONT = 26
_EMB = 128
_HID = 256
_N0, _N1, _N2 = 10, 10, 5
_NCOMBO = _N0 * _N1 * _N2

_BLK = 2048


def _combo_table_body(adep_ref, ades_ref, clus_ref, fc1w_ref, fc1b_ref,
                      fc2w_ref, fc2b_ref, out_ref):
    p0 = jnp.dot(adep_ref[...], fc1w_ref[0:_HID, :],
                 preferred_element_type=jnp.float32)
    p1 = jnp.dot(ades_ref[...], fc1w_ref[_HID:2 * _HID, :],
                 preferred_element_type=jnp.float32)
    p2 = jnp.dot(clus_ref[...], fc1w_ref[2 * _HID:3 * _HID, :],
                 preferred_element_type=jnp.float32)
    r = lax.broadcasted_iota(jnp.int32, (_NCOMBO, 1), 0)
    i0 = r // (_N1 * _N2)
    i1 = (r // _N2) % _N1
    i2 = r % _N2
    oh0 = (i0 == lax.broadcasted_iota(jnp.int32, (_NCOMBO, _N0), 1)
           ).astype(jnp.float32)
    oh1 = (i1 == lax.broadcasted_iota(jnp.int32, (_NCOMBO, _N1), 1)
           ).astype(jnp.float32)
    oh2 = (i2 == lax.broadcasted_iota(jnp.int32, (_NCOMBO, _N2), 1)
           ).astype(jnp.float32)
    pre = (jnp.dot(oh0, p0, preferred_element_type=jnp.float32)
           + jnp.dot(oh1, p1, preferred_element_type=jnp.float32)
           + jnp.dot(oh2, p2, preferred_element_type=jnp.float32)
           + fc1b_ref[...])
    h = jnp.maximum(pre, 0.0)
    out_ref[...] = (jnp.dot(h, fc2w_ref[...],
                            preferred_element_type=jnp.float32)
                    + fc2b_ref[...])


def _combo_table(adep_tab, ades_tab, cluster_tab, fc1_W, fc1_b, fc2_W, fc2_b):
    return pl.pallas_call(
        _combo_table_body,
        out_shape=jax.ShapeDtypeStruct((_NCOMBO, _EMB), jnp.float32),
    )(adep_tab, ades_tab, cluster_tab, fc1_W,
      fc1_b.reshape(1, _EMB), fc2_W, fc2_b.reshape(1, _EMB))


_NC, _NS = 2, 16
_NW = _NC * _NS
_BPW = _B // _NW
_GRP = _BPW // 16
_NSTREAM = _BPW // 128


def _sc_gather(cat_flat, table):
    mesh = plsc.VectorSubcoreMesh(core_axis_name="c", subcore_axis_name="s")

    @functools.partial(
        pl.kernel,
        out_type=jax.ShapeDtypeStruct((_B, _EMB), jnp.float32),
        mesh=mesh,
        scratch_types=[
            pltpu.VMEM((_BPW,), jnp.int32),
            pltpu.VMEM((_BPW,), jnp.int32),
            pltpu.VMEM((_BPW,), jnp.int32),
            pltpu.VMEM((_NSTREAM, 128), jnp.int32),
            pltpu.VMEM((_BPW, _EMB), jnp.float32),
            pltpu.SemaphoreType.DMA,
        ],
    )
    def run(cat_hbm, table_hbm, out_hbm, c0_v, c1_v, c2_v, idx_v, rows_v, sem):
        wid = lax.axis_index("s") * _NC + lax.axis_index("c")
        base = wid * _BPW
        pltpu.sync_copy(cat_hbm.at[pl.ds(base, _BPW)], c0_v)
        pltpu.sync_copy(cat_hbm.at[pl.ds(_B + base, _BPW)], c1_v)
        pltpu.sync_copy(cat_hbm.at[pl.ds(2 * _B + base, _BPW)], c2_v)
        for g in range(_GRP):
            s = pl.ds(g * 16, 16)
            combo = c0_v[s] * (_N1 * _N2) + c1_v[s] * _N2 + c2_v[s]
            idx_v[g // 8, pl.ds((g % 8) * 16, 16)] = combo
        copies = [
            pltpu.async_copy(table_hbm.at[idx_v.at[j]],
                             rows_v.at[pl.ds(j * 128, 128)], sem)
            for j in range(_NSTREAM)
        ]
        for c in copies:
            c.wait()
        pltpu.sync_copy(rows_v, out_hbm.at[pl.ds(base, _BPW)])

    return run(cat_flat, table)


def _wide_add_body(cont_ref, widew_ref, wideb_ref, deep_ref, out_ref):
    out_ref[...] = (jnp.dot(cont_ref[...], widew_ref[...],
                            preferred_element_type=jnp.float32)
                    + wideb_ref[...] + deep_ref[...])


def _wide_add(continuous_attrs, wide_W, wide_b, deep_rows):
    return pl.pallas_call(
        _wide_add_body,
        grid=(_B // _BLK,),
        in_specs=[
            pl.BlockSpec((_BLK, _CONT), lambda i: (i, 0)),
            pl.BlockSpec((_CONT, _EMB), lambda i: (0, 0)),
            pl.BlockSpec((1, _EMB), lambda i: (0, 0)),
            pl.BlockSpec((_BLK, _EMB), lambda i: (i, 0)),
        ],
        out_specs=pl.BlockSpec((_BLK, _EMB), lambda i: (i, 0)),
        out_shape=jax.ShapeDtypeStruct((_B, _EMB), jnp.float32),
    )(continuous_attrs, wide_W, wide_b.reshape(1, _EMB), deep_rows)


def kernel(continuous_attrs, categorical_attrs, wide_W, wide_b, adep_tab,
           ades_tab, cluster_tab, fc1_W, fc1_b, fc2_W, fc2_b):
    cat = jnp.asarray(categorical_attrs, jnp.int32)
    cat_flat = cat.T.reshape(-1)
    table = _combo_table(adep_tab, ades_tab, cluster_tab,
                         fc1_W, fc1_b, fc2_W, fc2_b)
    deep_rows = _sc_gather(cat_flat, table)
    return _wide_add(continuous_attrs, wide_W, wide_b, deep_rows)
